# baseline (speedup 1.0000x reference)
import functools
import math

import jax
import jax.numpy as jnp
from jax import lax
from jax.experimental import pallas as pl
from jax.experimental.pallas import tpu as pltpu
from jax.experimental.pallas import tpu_sc as plsc

NUM_BUCKETS = 32
MAX_DISTANCE = 128
N_HEADS = 16
SEQ = 2048
TBL = 4096
UNROLL = 8


def _table_body(w_ref, t_ref):
    x = lax.broadcasted_iota(jnp.int32, (1, TBL), 1)
    d = x - (SEQ - 1)

    num_buckets = NUM_BUCKETS // 2
    max_exact = num_buckets // 2
    rel_buckets = jnp.where(d > 0, num_buckets, 0)
    rp = jnp.abs(d)
    is_small = rp < max_exact
    rp_safe = jnp.maximum(rp, 1).astype(jnp.float32)
    large = max_exact + (
        jnp.log(rp_safe / max_exact)
        / math.log(MAX_DISTANCE / max_exact)
        * (num_buckets - max_exact)
    ).astype(jnp.int32)
    large = jnp.minimum(large, num_buckets - 1)
    bucket = rel_buckets + jnp.where(is_small, rp, large)

    bidx = lax.broadcasted_iota(jnp.int32, (NUM_BUCKETS, TBL), 0)
    onehot = (bidx == bucket).astype(jnp.float32)
    t_ref[...] = lax.dot_general(
        w_ref[...], onehot, (((0,), (0,)), ((), ())),
        preferred_element_type=jnp.float32,
        precision=lax.Precision.HIGHEST,
    )


_build_table = pl.pallas_call(
    _table_body,
    out_shape=jax.ShapeDtypeStruct((N_HEADS, TBL), jnp.float32),
)


_sc_mesh = plsc.VectorSubcoreMesh(core_axis_name="c", subcore_axis_name="s")


@functools.partial(
    pl.kernel,
    out_type=jax.ShapeDtypeStruct((1, N_HEADS, SEQ, SEQ), jnp.float32),
    mesh=_sc_mesh,
    scratch_types=[
        pltpu.VMEM((TBL,), jnp.float32),
        pltpu.VMEM((2, 8, SEQ), jnp.float32),
        pltpu.SemaphoreType.DMA,
    ],
    compiler_params=pltpu.CompilerParams(needs_layout_passes=False),
)
def _expand(t_hbm, out_hbm, tbl_v, buf_v, sem):
    core = lax.axis_index("c")
    sub = lax.axis_index("s")
    h = sub
    base_row = core * (SEQ // 2)
    ntr = SEQ // 2 // 8

    pltpu.sync_copy(t_hbm.at[pl.ds(pl.multiple_of(h * TBL, 8), TBL)], tbl_v)

    iota = lax.iota(jnp.int32, 16)

    def make_dma(t):
        slot = lax.bitwise_and(t, 1)
        row0 = pl.multiple_of(base_row + t * 8, 8)
        return pltpu.make_async_copy(
            buf_v.at[slot], out_hbm.at[0, h, pl.ds(row0, 8)], sem
        )

    def tile_block(t, _):
        slot = lax.bitwise_and(t, 1)
        row0 = base_row + t * 8

        @pl.when(t >= 2)
        def _():
            make_dma(t - 2).wait()

        for a in range(8):
            idx0 = ((SEQ - 1) - row0 - a) + iota

            @plsc.parallel_loop(0, SEQ // (UNROLL * 16), unroll=8)
            def _(cc, a=a, idx0=idx0, slot=slot):
                base = cc * (UNROLL * 16)
                vals = []
                for u in range(UNROLL):
                    idx = idx0 + (base + u * 16)
                    vals.append(plsc.load_gather(tbl_v, [idx]))
                for u in range(UNROLL):
                    col = pl.multiple_of(base + u * 16, 16)
                    buf_v[slot, a, pl.ds(col, 16)] = vals[u]

        make_dma(t).start()
        return 0

    lax.fori_loop(0, ntr, tile_block, 0)
    make_dma(ntr - 2).wait()
    make_dma(ntr - 1).wait()


def _copy_body(q_ref, k_ref, v_ref, qo_ref, ko_ref, vo_ref):
    qo_ref[...] = q_ref[...]
    ko_ref[...] = k_ref[...]
    vo_ref[...] = v_ref[...]


def _passthrough(q, k, v):
    blk = pl.BlockSpec((1, 1024, 1024), lambda b, i: (b, i, 0))
    return pl.pallas_call(
        _copy_body,
        grid=(2, 2),
        in_specs=[blk, blk, blk],
        out_specs=[blk, blk, blk],
        out_shape=[jax.ShapeDtypeStruct(q.shape, q.dtype)] * 3,
    )(q, k, v)


@jax.jit
def _bias(w):
    table = _build_table(w)
    return _expand(table.reshape(N_HEADS * TBL))


def kernel(q, k, v, W):
    qc, kc, vc = _passthrough(q, k, v)
    return (qc, kc, vc, _bias(W))


# speedup vs baseline: 1.0085x; 1.0085x over previous
import functools
import math

import jax
import jax.numpy as jnp
from jax import lax
from jax.experimental import pallas as pl
from jax.experimental.pallas import tpu as pltpu
from jax.experimental.pallas import tpu_sc as plsc

NUM_BUCKETS = 32
MAX_DISTANCE = 128
N_HEADS = 16
SEQ = 2048
TBL = 4096
UNROLL = 8


def _table_body(w_ref, t_ref):
    x = lax.broadcasted_iota(jnp.int32, (1, TBL), 1)
    d = x - (SEQ - 1)

    num_buckets = NUM_BUCKETS // 2
    max_exact = num_buckets // 2
    rel_buckets = jnp.where(d > 0, num_buckets, 0)
    rp = jnp.abs(d)
    is_small = rp < max_exact
    rp_safe = jnp.maximum(rp, 1).astype(jnp.float32)
    large = max_exact + (
        jnp.log(rp_safe / max_exact)
        / math.log(MAX_DISTANCE / max_exact)
        * (num_buckets - max_exact)
    ).astype(jnp.int32)
    large = jnp.minimum(large, num_buckets - 1)
    bucket = rel_buckets + jnp.where(is_small, rp, large)

    bidx = lax.broadcasted_iota(jnp.int32, (NUM_BUCKETS, TBL), 0)
    onehot = (bidx == bucket).astype(jnp.float32)
    t_ref[...] = lax.dot_general(
        w_ref[...], onehot, (((0,), (0,)), ((), ())),
        preferred_element_type=jnp.float32,
        precision=lax.Precision.HIGHEST,
    )


_build_table = pl.pallas_call(
    _table_body,
    out_shape=jax.ShapeDtypeStruct((N_HEADS, TBL), jnp.float32),
)


_sc_mesh = plsc.VectorSubcoreMesh(core_axis_name="c", subcore_axis_name="s")


@functools.partial(
    pl.kernel,
    out_type=jax.ShapeDtypeStruct((1, N_HEADS, SEQ, SEQ), jnp.float32),
    mesh=_sc_mesh,
    scratch_types=[
        pltpu.VMEM((TBL,), jnp.float32),
        pltpu.VMEM((2, 8, SEQ), jnp.float32),
        pltpu.SemaphoreType.DMA,
    ],
    compiler_params=pltpu.CompilerParams(needs_layout_passes=False),
)
def _expand(t_hbm, out_hbm, tbl_v, buf_v, sem):
    core = lax.axis_index("c")
    sub = lax.axis_index("s")
    h = sub
    base_row = core * (SEQ // 2)
    ntr = SEQ // 2 // 8

    pltpu.sync_copy(t_hbm.at[pl.ds(pl.multiple_of(h * TBL, 8), TBL)], tbl_v)

    iota = lax.iota(jnp.int32, 16)

    def make_dma(t):
        slot = lax.bitwise_and(t, 1)
        row0 = pl.multiple_of(base_row + t * 8, 8)
        return pltpu.make_async_copy(
            buf_v.at[slot], out_hbm.at[0, h, pl.ds(row0, 8)], sem
        )

    def tile_block(t, _):
        slot = lax.bitwise_and(t, 1)
        row0 = base_row + t * 8

        @pl.when(t >= 2)
        def _():
            make_dma(t - 2).wait()

        for a in range(8):
            idx0 = ((SEQ - 1) - row0 - a) + iota

            @plsc.parallel_loop(0, SEQ // (UNROLL * 16), unroll=4)
            def _(cc, a=a, idx0=idx0, slot=slot):
                base = cc * (UNROLL * 16)
                vals = []
                for u in range(UNROLL):
                    idx = idx0 + (base + u * 16)
                    vals.append(plsc.load_gather(tbl_v, [idx]))
                for u in range(UNROLL):
                    col = pl.multiple_of(base + u * 16, 16)
                    buf_v[slot, a, pl.ds(col, 16)] = vals[u]

        make_dma(t).start()
        return 0

    lax.fori_loop(0, ntr, tile_block, 0)
    make_dma(ntr - 2).wait()
    make_dma(ntr - 1).wait()


def _copy_body(q_ref, k_ref, v_ref, qo_ref, ko_ref, vo_ref):
    qo_ref[...] = q_ref[...]
    ko_ref[...] = k_ref[...]
    vo_ref[...] = v_ref[...]


def _passthrough(q, k, v):
    blk = pl.BlockSpec((1, 1024, 1024), lambda b, i: (b, i, 0))
    return pl.pallas_call(
        _copy_body,
        grid=(2, 2),
        in_specs=[blk, blk, blk],
        out_specs=[blk, blk, blk],
        out_shape=[jax.ShapeDtypeStruct(q.shape, q.dtype)] * 3,
    )(q, k, v)


@jax.jit
def _bias(w):
    table = _build_table(w)
    return _expand(table.reshape(N_HEADS * TBL))


def kernel(q, k, v, W):
    qc, kc, vc = _passthrough(q, k, v)
    return (qc, kc, vc, _bias(W))
